# baseline (speedup 1.0000x reference)
import functools

import numpy as np
import jax
import jax.numpy as jnp
from jax import lax
from jax.experimental import pallas as pl
from jax.experimental.pallas import tpu as pltpu
from jax.experimental.pallas import tpu_sc as plsc

D_M = 384
NH = 8
NL = 4
NP = 4
HD = 48
SIZES = ((64, 64), (32, 32), (16, 16), (8, 8))
S_L = tuple(h * w for h, w in SIZES)
LVL_OFF = (0, 4096, 5120, 5376)
S_TOT = 5440
NCOL = NH * NL * NP

_col = np.arange(NCOL)
_lvl = (_col // NP) % NL
_h_c = _col // (NL * NP)
_WL = np.array([SIZES[l][1] for l in _lvl], np.float32)
_HL = np.array([SIZES[l][0] for l in _lvl], np.float32)
_WL_M1 = (_WL - 1.0).reshape(1, NCOL)
_HL_M1 = (_HL - 1.0).reshape(1, NCOL)
_COLBASE = (_h_c * S_TOT + np.array(LVL_OFF)[_lvl]).astype(np.int32).reshape(1, NCOL)
_WL_I = _WL.astype(np.int32).reshape(1, NCOL)
_GMASK = (_col[:, None] // 16 == _col[None, :] // 16).astype(np.float32)

RBLK = 512

NW = 32
LANES = 16


def _prep_body(q_ref, refp_ref, wx_ref, wy_ref, bx_ref, by_ref, wa_ref, ba_ref,
               gm_ref, wlm1_ref, hlm1_ref, colb_ref, wli_ref,
               attn_ref, i00_ref,
               w00_ref, w01_ref, w10_ref, w11_ref, *, q_per_blk):
    q = q_ref[...]
    ox = jnp.tanh(jnp.dot(q, wx_ref[...], preferred_element_type=jnp.float32) + bx_ref[...])
    oy = jnp.tanh(jnp.dot(q, wy_ref[...], preferred_element_type=jnp.float32) + by_ref[...])
    rx = refp_ref[:, 0:1]
    ry = refp_ref[:, 1:2]
    px = (rx + 0.5 * ox) * wlm1_ref[...]
    py = (ry + 0.5 * oy) * hlm1_ref[...]
    x0 = jnp.floor(px)
    fx = px - x0
    y0 = jnp.floor(py)
    fy = py - y0
    bx = jnp.clip(x0, 0.0, wlm1_ref[...] - 1.0)
    by = jnp.clip(y0, 0.0, hlm1_ref[...] - 1.0)
    wx0 = jnp.where(bx == x0, 1.0 - fx, 0.0) + jnp.where(bx == x0 + 1.0, fx, 0.0)
    wx1 = jnp.where(bx + 1.0 == x0, 1.0 - fx, 0.0) + jnp.where(bx == x0, fx, 0.0)
    wy0 = jnp.where(by == y0, 1.0 - fy, 0.0) + jnp.where(by == y0 + 1.0, fy, 0.0)
    wy1 = jnp.where(by + 1.0 == y0, 1.0 - fy, 0.0) + jnp.where(by == y0, fy, 0.0)
    al = jnp.dot(q, wa_ref[...], preferred_element_type=jnp.float32) + ba_ref[...]
    m = jnp.max(al, axis=1, keepdims=True)
    e = jnp.exp(al - m)
    att = e / jnp.dot(e, gm_ref[...], preferred_element_type=jnp.float32)
    attn_ref[...] = att
    b = pl.program_id(0) // q_per_blk
    base = b * (NH * S_TOT)
    i00 = (base + colb_ref[...]
           + by.astype(jnp.int32) * wli_ref[...] + bx.astype(jnp.int32))
    i00_ref[...] = i00
    w00_ref[...] = wy0 * wx0 * att
    w01_ref[...] = wy0 * wx1 * att
    w10_ref[...] = wy1 * wx0 * att
    w11_ref[...] = wy1 * wx1 * att


def _make_prep(BQ, q_per_blk):
    full = lambda i: (0, 0)
    row = lambda i: (i, 0)
    return pl.pallas_call(
        functools.partial(_prep_body, q_per_blk=q_per_blk),
        grid=(BQ // RBLK,),
        in_specs=[
            pl.BlockSpec((RBLK, D_M), row),
            pl.BlockSpec((RBLK, 2), row),
            pl.BlockSpec((D_M, NCOL), full),
            pl.BlockSpec((D_M, NCOL), full),
            pl.BlockSpec((1, NCOL), full),
            pl.BlockSpec((1, NCOL), full),
            pl.BlockSpec((D_M, NCOL), full),
            pl.BlockSpec((1, NCOL), full),
            pl.BlockSpec((NCOL, NCOL), full),
            pl.BlockSpec((1, NCOL), full),
            pl.BlockSpec((1, NCOL), full),
            pl.BlockSpec((1, NCOL), full),
            pl.BlockSpec((1, NCOL), full),
        ],
        out_specs=[pl.BlockSpec((RBLK, NCOL), row) for _ in range(6)],
        out_shape=[jax.ShapeDtypeStruct((BQ, NCOL), jnp.float32),
                   jax.ShapeDtypeStruct((BQ, NCOL), jnp.int32)]
                  + [jax.ShapeDtypeStruct((BQ, NCOL), jnp.float32)
                     for _ in range(4)],
    )


def _build_table(feats):
    B = feats[0].shape[0]
    parts = []
    for (Hl, Wl), f in zip(SIZES, feats):
        S = Hl * Wl
        t = f.reshape(B, NH, HD, S).transpose(0, 1, 3, 2)

        def sh(n):
            return jnp.pad(t, ((0, 0), (0, 0), (0, n), (0, 0)))[:, :, n:n + S]

        y0 = jnp.stack([t, sh(1)], axis=4)
        y1 = jnp.stack([sh(Wl), sh(Wl + 1)], axis=4)
        parts.append(jnp.concatenate([y0, y1], axis=3))
    tbl = jnp.concatenate(parts, axis=2).astype(jnp.bfloat16)
    tbl = lax.bitcast_convert_type(tbl, jnp.int32)
    return tbl.reshape(B * NH * S_TOT, 2 * HD)


def _sc_body(BQ, table_hbm, i0_hbm,
             w0_hbm, w1_hbm, w2_hbm, w3_hbm, out_hbm,
             ibufA, ibufB, gbufA, gbufB, wbufA, wbufB, obuf,
             semGA, semGB, semWA, semWB, semIA, semIB):
    rows = BQ // NW
    wid = lax.axis_index("s") * 2 + lax.axis_index("c")
    r0 = wid * rows
    ibufs = (ibufA, ibufB)
    gbufs = (gbufA, gbufB)
    wbufs = (wbufA, wbufB)
    semG = (semGA, semGB)
    semW = (semWA, semWB)
    semI = (semIA, semIB)
    w_hbms = (w0_hbm, w1_hbm, w2_hbm, w3_hbm)

    def i_copy(rl, b):
        return pltpu.make_async_copy(
            i0_hbm.at[pl.ds((r0 + rl) * NCOL, NCOL)], ibufs[b], semI[b])

    def g_copy(rl, b):
        return pltpu.make_async_copy(
            table_hbm.at[ibufs[b]], gbufs[b], semG[b])

    def w_copy(rl, b, c):
        return pltpu.make_async_copy(
            w_hbms[c].at[pl.ds((r0 + rl) * NCOL, NCOL)],
            wbufs[b].at[pl.ds(c * NCOL, NCOL)], semW[b])

    i_copy(0, 0).start()
    i_copy(0, 0).wait()
    i_copy(1, 1).start()
    g_copy(0, 0).start()
    for c in range(4):
        w_copy(0, 0, c).start()

    def outer(i, carry):
        g = i * 2
        for par in range(2):
            rl = g + par
            b = par
            nb = 1 - par

            @pl.when(rl + 1 < rows)
            def _():
                i_copy(rl + 1, nb).wait()
                g_copy(rl + 1, nb).start()
                for c in range(4):
                    w_copy(rl + 1, nb, c).start()

            g_copy(rl, b).wait()
            for c in range(4):
                w_copy(rl, b, c).wait()

            @pl.when(rl + 2 < rows)
            def _():
                i_copy(rl + 2, b).start()

            gb = gbufs[b]
            wb = wbufs[b]

            def hbody(hh, c2):
                t0 = hh * 16
                acc = [jnp.zeros((LANES,), jnp.float32) for _ in range(3)]
                wvecs = [wb[pl.ds(c * NCOL + t0, LANES)] for c in range(4)]
                for t16 in range(16):
                    t = t0 + t16
                    for j in range(2):
                        wl = jnp.full((LANES,), wvecs[2 * j][t16],
                                      dtype=jnp.float32)
                        wr = jnp.full((LANES,), wvecs[2 * j + 1][t16],
                                      dtype=jnp.float32)
                        for k in range(3):
                            word = gb[t, pl.ds(j * HD + k * LANES, LANES)]
                            ev = lax.bitcast_convert_type(
                                word << 16, jnp.float32)
                            od = lax.bitcast_convert_type(
                                word & jnp.int32(-65536), jnp.float32)
                            acc[k] = acc[k] + wl * ev + wr * od
                for k in range(3):
                    obuf[pl.ds(hh * HD + k * LANES, LANES)] = acc[k]
                return c2

            lax.fori_loop(0, NH, hbody, 0)
            for j in range(3):
                pltpu.sync_copy(
                    obuf.at[pl.ds(j * 128, 128)],
                    out_hbm.at[pl.ds(j * (BQ * 128) + (r0 + rl) * 128, 128)])
        return carry

    lax.fori_loop(0, rows // 2, outer, 0)


def _make_sc(BQ):
    mesh = plsc.VectorSubcoreMesh(core_axis_name="c", subcore_axis_name="s")
    return pl.kernel(
        functools.partial(_sc_body, BQ),
        out_type=jax.ShapeDtypeStruct((BQ * D_M,), jnp.float32),
        mesh=mesh,
        compiler_params=pltpu.CompilerParams(use_tc_tiling_on_sc=False),
        scratch_types=[
            pltpu.VMEM((NCOL,), jnp.int32),
            pltpu.VMEM((NCOL,), jnp.int32),
            pltpu.VMEM((NCOL, 2 * HD), jnp.int32),
            pltpu.VMEM((NCOL, 2 * HD), jnp.int32),
            pltpu.VMEM((4 * NCOL,), jnp.float32),
            pltpu.VMEM((4 * NCOL,), jnp.float32),
            pltpu.VMEM((D_M,), jnp.float32),
            pltpu.SemaphoreType.DMA,
            pltpu.SemaphoreType.DMA,
            pltpu.SemaphoreType.DMA,
            pltpu.SemaphoreType.DMA,
            pltpu.SemaphoreType.DMA,
            pltpu.SemaphoreType.DMA,
        ],
    )


def _oproj_body(a0_ref, a1_ref, a2_ref, w0_ref, w1_ref, w2_ref, b_ref, o_ref):
    a0 = a0_ref[...].reshape(RBLK, 128)
    a1 = a1_ref[...].reshape(RBLK, 128)
    a2 = a2_ref[...].reshape(RBLK, 128)
    acc = jnp.dot(a0, w0_ref[...], preferred_element_type=jnp.float32)
    acc += jnp.dot(a1, w1_ref[...], preferred_element_type=jnp.float32)
    acc += jnp.dot(a2, w2_ref[...], preferred_element_type=jnp.float32)
    o_ref[...] = acc + b_ref[...]


def _make_oproj(BQ):
    nblk = BQ // RBLK
    return pl.pallas_call(
        _oproj_body,
        grid=(nblk,),
        in_specs=[
            pl.BlockSpec((RBLK * 128,), lambda i: (i,)),
            pl.BlockSpec((RBLK * 128,), lambda i: (i + nblk,)),
            pl.BlockSpec((RBLK * 128,), lambda i: (i + 2 * nblk,)),
            pl.BlockSpec((128, D_M), lambda i: (0, 0)),
            pl.BlockSpec((128, D_M), lambda i: (1, 0)),
            pl.BlockSpec((128, D_M), lambda i: (2, 0)),
            pl.BlockSpec((1, D_M), lambda i: (0, 0)),
        ],
        out_specs=pl.BlockSpec((RBLK, D_M), lambda i: (i, 0)),
        out_shape=jax.ShapeDtypeStruct((BQ, D_M), jnp.float32),
    )


def kernel(query, feat0, feat1, feat2, feat3, reference_points,
           W_off, b_off, W_attn, b_attn, W_out, b_out):
    B, Q, _ = query.shape
    BQ = B * Q
    q2 = query.reshape(BQ, D_M)
    refp2 = reference_points.reshape(BQ, 2)
    Wx = W_off[:, 0::2]
    Wy = W_off[:, 1::2]
    bx = b_off[0::2].reshape(1, NCOL)
    by = b_off[1::2].reshape(1, NCOL)
    ba = b_attn.reshape(1, NCOL)

    attn, i00, w00, w01, w10, w11 = _make_prep(BQ, Q // RBLK)(
        q2, refp2, Wx, Wy, bx, by, W_attn, ba,
        jnp.asarray(_GMASK), jnp.asarray(_WL_M1), jnp.asarray(_HL_M1),
        jnp.asarray(_COLBASE), jnp.asarray(_WL_I))

    table = _build_table([feat0, feat1, feat2, feat3])

    agg = _make_sc(BQ)(table, i00.reshape(BQ * NCOL),
                       w00.reshape(BQ * NCOL), w01.reshape(BQ * NCOL),
                       w10.reshape(BQ * NCOL), w11.reshape(BQ * NCOL))
    out = _make_oproj(BQ)(agg, agg, agg, W_out, W_out, W_out,
                          b_out.reshape(1, D_M))
    return out.reshape(B, Q, D_M), attn.reshape(B, Q, NH, NL, NP)


# speedup vs baseline: 1.2548x; 1.2548x over previous
import functools

import numpy as np
import jax
import jax.numpy as jnp
from jax import lax
from jax.experimental import pallas as pl
from jax.experimental.pallas import tpu as pltpu
from jax.experimental.pallas import tpu_sc as plsc

D_M = 384
NH = 8
NL = 4
NP = 4
HD = 48
SIZES = ((64, 64), (32, 32), (16, 16), (8, 8))
S_L = tuple(h * w for h, w in SIZES)
LVL_OFF = (0, 4096, 5120, 5376)
S_TOT = 5440
NCOL = NH * NL * NP

_col = np.arange(NCOL)
_lvl = (_col // NP) % NL
_h_c = _col // (NL * NP)
_WL = np.array([SIZES[l][1] for l in _lvl], np.float32)
_HL = np.array([SIZES[l][0] for l in _lvl], np.float32)
_WL_M1 = (_WL - 1.0).reshape(1, NCOL)
_HL_M1 = (_HL - 1.0).reshape(1, NCOL)
_COLBASE = (_h_c * S_TOT + np.array(LVL_OFF)[_lvl]).astype(np.int32).reshape(1, NCOL)
_WL_I = _WL.astype(np.int32).reshape(1, NCOL)
_GMASK = (_col[:, None] // 16 == _col[None, :] // 16).astype(np.float32)

RBLK = 512

NW = 32
LANES = 16


def _prep_body(q_ref, refp_ref, wx_ref, wy_ref, bx_ref, by_ref, wa_ref, ba_ref,
               gm_ref, wlm1_ref, hlm1_ref, colb_ref, wli_ref,
               attn_ref, i00_ref, i10_ref,
               w00_ref, w01_ref, w10_ref, w11_ref, *, q_per_blk):
    q = q_ref[...]
    ox = jnp.tanh(jnp.dot(q, wx_ref[...], preferred_element_type=jnp.float32) + bx_ref[...])
    oy = jnp.tanh(jnp.dot(q, wy_ref[...], preferred_element_type=jnp.float32) + by_ref[...])
    rx = refp_ref[:, 0:1]
    ry = refp_ref[:, 1:2]
    px = (rx + 0.5 * ox) * wlm1_ref[...]
    py = (ry + 0.5 * oy) * hlm1_ref[...]
    x0 = jnp.floor(px)
    fx = px - x0
    y0 = jnp.floor(py)
    fy = py - y0
    bx = jnp.clip(x0, 0.0, wlm1_ref[...] - 1.0)
    by = jnp.clip(y0, 0.0, hlm1_ref[...] - 1.0)
    wx0 = jnp.where(bx == x0, 1.0 - fx, 0.0) + jnp.where(bx == x0 + 1.0, fx, 0.0)
    wx1 = jnp.where(bx + 1.0 == x0, 1.0 - fx, 0.0) + jnp.where(bx == x0, fx, 0.0)
    wy0 = jnp.where(by == y0, 1.0 - fy, 0.0) + jnp.where(by == y0 + 1.0, fy, 0.0)
    wy1 = jnp.where(by + 1.0 == y0, 1.0 - fy, 0.0) + jnp.where(by == y0, fy, 0.0)
    al = jnp.dot(q, wa_ref[...], preferred_element_type=jnp.float32) + ba_ref[...]
    m = jnp.max(al, axis=1, keepdims=True)
    e = jnp.exp(al - m)
    att = e / jnp.dot(e, gm_ref[...], preferred_element_type=jnp.float32)
    attn_ref[...] = att
    b = pl.program_id(0) // q_per_blk
    base = b * (NH * S_TOT)
    i00 = (base + colb_ref[...]
           + by.astype(jnp.int32) * wli_ref[...] + bx.astype(jnp.int32))
    i00_ref[...] = i00
    i10_ref[...] = i00 + wli_ref[...]
    w00_ref[...] = wy0 * wx0 * att
    w01_ref[...] = wy0 * wx1 * att
    w10_ref[...] = wy1 * wx0 * att
    w11_ref[...] = wy1 * wx1 * att


def _make_prep(BQ, q_per_blk):
    full = lambda i: (0, 0)
    row = lambda i: (i, 0)
    return pl.pallas_call(
        functools.partial(_prep_body, q_per_blk=q_per_blk),
        grid=(BQ // RBLK,),
        in_specs=[
            pl.BlockSpec((RBLK, D_M), row),
            pl.BlockSpec((RBLK, 2), row),
            pl.BlockSpec((D_M, NCOL), full),
            pl.BlockSpec((D_M, NCOL), full),
            pl.BlockSpec((1, NCOL), full),
            pl.BlockSpec((1, NCOL), full),
            pl.BlockSpec((D_M, NCOL), full),
            pl.BlockSpec((1, NCOL), full),
            pl.BlockSpec((NCOL, NCOL), full),
            pl.BlockSpec((1, NCOL), full),
            pl.BlockSpec((1, NCOL), full),
            pl.BlockSpec((1, NCOL), full),
            pl.BlockSpec((1, NCOL), full),
        ],
        out_specs=[pl.BlockSpec((RBLK, NCOL), row) for _ in range(7)],
        out_shape=[jax.ShapeDtypeStruct((BQ, NCOL), jnp.float32),
                   jax.ShapeDtypeStruct((BQ, NCOL), jnp.int32),
                   jax.ShapeDtypeStruct((BQ, NCOL), jnp.int32)]
                  + [jax.ShapeDtypeStruct((BQ, NCOL), jnp.float32)
                     for _ in range(4)],
    )


def _build_table(feats):
    B = feats[0].shape[0]
    parts = []
    for (Hl, Wl), f in zip(SIZES, feats):
        S = Hl * Wl
        t = f.reshape(B, NH, HD, S).transpose(0, 1, 3, 2)
        tp = jnp.pad(t, ((0, 0), (0, 0), (0, 1), (0, 0)))[:, :, 1:S + 1]
        parts.append(jnp.stack([t, tp], axis=4))
    tbl = jnp.concatenate(parts, axis=2).astype(jnp.bfloat16)
    tbl = lax.bitcast_convert_type(tbl, jnp.int32)
    return tbl.reshape(B * NH * S_TOT, HD)


def _sc_body(BQ, table_hbm, i0_hbm, i1_hbm,
             w0_hbm, w1_hbm, w2_hbm, w3_hbm, out_hbm,
             ibufA, ibufB, gbufA, gbufB, wbufA, wbufB, obuf,
             semGA, semGB, semWA, semWB, semIA, semIB, semOA, semOB):
    rows = BQ // NW
    wid = lax.axis_index("s") * 2 + lax.axis_index("c")
    r0 = wid * rows
    ibufs = (ibufA, ibufB)
    gbufs = (gbufA, gbufB)
    wbufs = (wbufA, wbufB)
    semG = (semGA, semGB)
    semW = (semWA, semWB)
    semI = (semIA, semIB)
    semO = (semOA, semOB)
    i_hbms = (i0_hbm, i1_hbm)
    w_hbms = (w0_hbm, w1_hbm, w2_hbm, w3_hbm)
    NI = 2

    def i_copy(rl, b, c):
        return pltpu.make_async_copy(
            i_hbms[c].at[pl.ds((r0 + rl) * NCOL, NCOL)],
            ibufs[b].at[pl.ds(c * NCOL, NCOL)], semI[b])

    def g_copy(rl, b, c):
        return pltpu.make_async_copy(
            table_hbm.at[ibufs[b].at[pl.ds(c * NCOL, NCOL)]],
            gbufs[b].at[pl.ds(c * NCOL, NCOL), :], semG[b])

    def w_copy(rl, b, c):
        return pltpu.make_async_copy(
            w_hbms[c].at[pl.ds((r0 + rl) * NCOL, NCOL)],
            wbufs[b].at[pl.ds(c * NCOL, NCOL)], semW[b])

    def o_copy(rl, b, j):
        return pltpu.make_async_copy(
            obuf.at[pl.ds(b * D_M + j * 128, 128)],
            out_hbm.at[pl.ds(j * (BQ * 128) + (r0 + rl) * 128, 128)],
            semO[b])

    for c in range(NI):
        i_copy(0, 0, c).start()
    for c in range(NI):
        i_copy(0, 0, c).wait()
    for c in range(NI):
        i_copy(1, 1, c).start()
    for c in range(NI):
        g_copy(0, 0, c).start()
    for c in range(4):
        w_copy(0, 0, c).start()

    def outer(i, carry):
        g = i * 2
        for par in range(2):
            rl = g + par
            b = par
            nb = 1 - par

            @pl.when(rl + 1 < rows)
            def _():
                for c in range(NI):
                    i_copy(rl + 1, nb, c).wait()
                for c in range(NI):
                    g_copy(rl + 1, nb, c).start()
                for c in range(4):
                    w_copy(rl + 1, nb, c).start()

            for c in range(NI):
                g_copy(rl, b, c).wait()
            for c in range(4):
                w_copy(rl, b, c).wait()

            @pl.when(rl + 2 < rows)
            def _():
                for c in range(NI):
                    i_copy(rl + 2, b, c).start()

            gb = gbufs[b]
            wb = wbufs[b]

            def hbody(hh, c2):
                t0 = hh * 16
                acc = [jnp.zeros((LANES,), jnp.float32) for _ in range(3)]
                wvecs = [wb[pl.ds(c * NCOL + t0, LANES)] for c in range(4)]
                for t16 in range(16):
                    t = t0 + t16
                    for j in range(2):
                        wl = jnp.full((LANES,), wvecs[2 * j][t16],
                                      dtype=jnp.float32)
                        wr = jnp.full((LANES,), wvecs[2 * j + 1][t16],
                                      dtype=jnp.float32)
                        for k in range(3):
                            word = gb[j * NCOL + t, pl.ds(k * LANES, LANES)]
                            ev = lax.bitcast_convert_type(
                                word << 16, jnp.float32)
                            od = lax.bitcast_convert_type(
                                word & jnp.int32(-65536), jnp.float32)
                            acc[k] = acc[k] + wl * ev + wr * od
                for k in range(3):
                    obuf[pl.ds(b * D_M + hh * HD + k * LANES, LANES)] = acc[k]
                return c2

            @pl.when(rl >= 2)
            def _():
                for j in range(3):
                    o_copy(rl - 2, b, j).wait()

            lax.fori_loop(0, NH, hbody, 0)
            for j in range(3):
                o_copy(rl, b, j).start()
        return carry

    lax.fori_loop(0, rows // 2, outer, 0)
    for j in range(3):
        o_copy(rows - 2, 0, j).wait()
    for j in range(3):
        o_copy(rows - 1, 1, j).wait()


def _make_sc(BQ):
    mesh = plsc.VectorSubcoreMesh(core_axis_name="c", subcore_axis_name="s")
    return pl.kernel(
        functools.partial(_sc_body, BQ),
        out_type=jax.ShapeDtypeStruct((BQ * D_M,), jnp.float32),
        mesh=mesh,
        compiler_params=pltpu.CompilerParams(use_tc_tiling_on_sc=False),
        scratch_types=[
            pltpu.VMEM((2 * NCOL,), jnp.int32),
            pltpu.VMEM((2 * NCOL,), jnp.int32),
            pltpu.VMEM((2 * NCOL, HD), jnp.int32),
            pltpu.VMEM((2 * NCOL, HD), jnp.int32),
            pltpu.VMEM((4 * NCOL,), jnp.float32),
            pltpu.VMEM((4 * NCOL,), jnp.float32),
            pltpu.VMEM((2 * D_M,), jnp.float32),
            pltpu.SemaphoreType.DMA,
            pltpu.SemaphoreType.DMA,
            pltpu.SemaphoreType.DMA,
            pltpu.SemaphoreType.DMA,
            pltpu.SemaphoreType.DMA,
            pltpu.SemaphoreType.DMA,
            pltpu.SemaphoreType.DMA,
            pltpu.SemaphoreType.DMA,
        ],
    )


def _oproj_body(a0_ref, a1_ref, a2_ref, w0_ref, w1_ref, w2_ref, b_ref, o_ref):
    a0 = a0_ref[...].reshape(RBLK, 128)
    a1 = a1_ref[...].reshape(RBLK, 128)
    a2 = a2_ref[...].reshape(RBLK, 128)
    acc = jnp.dot(a0, w0_ref[...], preferred_element_type=jnp.float32)
    acc += jnp.dot(a1, w1_ref[...], preferred_element_type=jnp.float32)
    acc += jnp.dot(a2, w2_ref[...], preferred_element_type=jnp.float32)
    o_ref[...] = acc + b_ref[...]


def _make_oproj(BQ):
    nblk = BQ // RBLK
    return pl.pallas_call(
        _oproj_body,
        grid=(nblk,),
        in_specs=[
            pl.BlockSpec((RBLK * 128,), lambda i: (i,)),
            pl.BlockSpec((RBLK * 128,), lambda i: (i + nblk,)),
            pl.BlockSpec((RBLK * 128,), lambda i: (i + 2 * nblk,)),
            pl.BlockSpec((128, D_M), lambda i: (0, 0)),
            pl.BlockSpec((128, D_M), lambda i: (1, 0)),
            pl.BlockSpec((128, D_M), lambda i: (2, 0)),
            pl.BlockSpec((1, D_M), lambda i: (0, 0)),
        ],
        out_specs=pl.BlockSpec((RBLK, D_M), lambda i: (i, 0)),
        out_shape=jax.ShapeDtypeStruct((BQ, D_M), jnp.float32),
    )


def kernel(query, feat0, feat1, feat2, feat3, reference_points,
           W_off, b_off, W_attn, b_attn, W_out, b_out):
    B, Q, _ = query.shape
    BQ = B * Q
    q2 = query.reshape(BQ, D_M)
    refp2 = reference_points.reshape(BQ, 2)
    Wx = W_off[:, 0::2]
    Wy = W_off[:, 1::2]
    bx = b_off[0::2].reshape(1, NCOL)
    by = b_off[1::2].reshape(1, NCOL)
    ba = b_attn.reshape(1, NCOL)

    attn, i00, i10, w00, w01, w10, w11 = _make_prep(BQ, Q // RBLK)(
        q2, refp2, Wx, Wy, bx, by, W_attn, ba,
        jnp.asarray(_GMASK), jnp.asarray(_WL_M1), jnp.asarray(_HL_M1),
        jnp.asarray(_COLBASE), jnp.asarray(_WL_I))

    table = _build_table([feat0, feat1, feat2, feat3])

    agg = _make_sc(BQ)(table,
                       i00.reshape(BQ * NCOL), i10.reshape(BQ * NCOL),
                       w00.reshape(BQ * NCOL), w01.reshape(BQ * NCOL),
                       w10.reshape(BQ * NCOL), w11.reshape(BQ * NCOL))
    out = _make_oproj(BQ)(agg, agg, agg, W_out, W_out, W_out,
                          b_out.reshape(1, D_M))
    return out.reshape(B, Q, D_M), attn.reshape(B, Q, NH, NL, NP)
